# baseline (speedup 1.0000x reference)
import jax
import jax.numpy as jnp
from jax import lax
from jax.experimental import pallas as pl
from jax.experimental.pallas import tpu as pltpu
from jax.experimental.pallas import tpu_sc as plsc

_NC = 2
_NS = 16
_NW = _NC * _NS
_B = 4096
_PW = _B // _NW
_D = 128
_H = _PW // 2
_MARGIN = 1.0


def _rsqrt(x):
    i = lax.bitcast_convert_type(x, jnp.int32)
    i = jnp.int32(0x5F3759DF) - lax.shift_right_logical(i, 1)
    y = lax.bitcast_convert_type(i, jnp.float32)
    for _ in range(3):
        y = y * (jnp.float32(1.5) - jnp.float32(0.5) * x * y * y)
    return y


def _body(ent, rel, idx_hbm, out, ixf, rpe, rne, rr, lv, sem_a, sem_b):
    wid = lax.axis_index("s") * _NC + lax.axis_index("c")

    pltpu.sync_copy(idx_hbm.at[wid], ixf)

    sems = (sem_a, sem_b)
    copies = []
    for h in range(2):
        sl = pl.ds(h * 2 * _H, 2 * _H)
        copies.append((
            pltpu.async_copy(ent.at[ixf.at[h, 0]], rpe.at[sl], sems[h]),
            pltpu.async_copy(ent.at[ixf.at[h, 1]], rne.at[sl], sems[h]),
            pltpu.async_copy(rel.at[ixf.at[h, 2]], rr.at[sl], sems[h]),
        ))

    lane = lax.iota(jnp.int32, 16)
    zero = jnp.zeros((16,), jnp.float32)
    two = jnp.float32(2.0)
    eps_n = jnp.float32(1e-24)
    eps_d = jnp.float32(1e-12)

    loss = zero
    for h in range(2):
        for c in copies[h]:
            c.wait()
        for gg in range(_H // 16):
            base = jnp.int32(h * 2 * _H + gg * 16)
            r_ph = lane + base
            r_pt = r_ph + jnp.int32(_H)
            r_pr = r_ph
            r_nr = r_pt

            def dim_body(d, acc):
                (psh, psr, pst, pshr, psht, psrt,
                 nsh, nsr, nst, nshr, nsht, nsrt) = acc
                dv = (jnp.full((16,), d, jnp.int32) + lane) \
                    & jnp.int32(_D - 1)
                hh = plsc.load_gather(rpe, [r_ph, dv])
                rv = plsc.load_gather(rr, [r_pr, dv])
                tt = plsc.load_gather(rpe, [r_pt, dv])
                psh = psh + hh * hh
                psr = psr + rv * rv
                pst = pst + tt * tt
                pshr = pshr + hh * rv
                psht = psht + hh * tt
                psrt = psrt + rv * tt
                hh = plsc.load_gather(rne, [r_ph, dv])
                rv = plsc.load_gather(rr, [r_nr, dv])
                tt = plsc.load_gather(rne, [r_pt, dv])
                nsh = nsh + hh * hh
                nsr = nsr + rv * rv
                nst = nst + tt * tt
                nshr = nshr + hh * rv
                nsht = nsht + hh * tt
                nsrt = nsrt + rv * tt
                return (psh, psr, pst, pshr, psht, psrt,
                        nsh, nsr, nst, nshr, nsht, nsrt)

            (psh, psr, pst, pshr, psht, psrt,
             nsh, nsr, nst, nshr, nsht, nsrt) = lax.fori_loop(
                0, _D, dim_body, (zero,) * 12, unroll=8)

            ih = _rsqrt(jnp.maximum(psh, eps_n))
            it = _rsqrt(jnp.maximum(pst, eps_n))
            sp = (psh * ih * ih + psr + pst * it * it
                  + two * (ih * pshr - ih * it * psht - it * psrt)) + eps_d
            ih = _rsqrt(jnp.maximum(nsh, eps_n))
            it = _rsqrt(jnp.maximum(nst, eps_n))
            sn = (nsh * ih * ih + nsr + nst * it * it
                  + two * (ih * nshr - ih * it * nsht - it * nsrt)) + eps_d
            dp = sp * _rsqrt(sp)
            dn = sn * _rsqrt(sn)
            loss = loss + jnp.maximum(dp - dn + jnp.float32(_MARGIN),
                                      jnp.float32(0.0))

    lv[...] = loss
    pltpu.sync_copy(lv, out.at[wid])


@jax.jit
def _transe_loss(entity_emb, relation_emb, idx_all):
    mesh = plsc.VectorSubcoreMesh(core_axis_name="c", subcore_axis_name="s")
    f = pl.kernel(
        _body,
        out_type=jax.ShapeDtypeStruct((_NW, 16), jnp.float32),
        mesh=mesh,
        compiler_params=pltpu.CompilerParams(needs_layout_passes=False),
        scratch_types=[
            pltpu.VMEM((2, 3, _PW), jnp.int32),
            pltpu.VMEM((2 * _PW, _D), jnp.float32),
            pltpu.VMEM((2 * _PW, _D), jnp.float32),
            pltpu.VMEM((2 * _PW, _D), jnp.float32),
            pltpu.VMEM((16,), jnp.float32),
            pltpu.SemaphoreType.DMA,
            pltpu.SemaphoreType.DMA,
        ],
    )
    partials = f(entity_emb, relation_emb, idx_all)
    return jnp.sum(partials)


def kernel(entity_emb, relation_emb, pos_heads, pos_rels, pos_tails,
           neg_heads, neg_rels, neg_tails):
    ph, pr, pt, nh, nr, nt = (
        x.astype(jnp.int32).reshape(_NW, 2, _H)
        for x in (pos_heads, pos_rels, pos_tails,
                  neg_heads, neg_rels, neg_tails))
    s0 = jnp.concatenate([ph, pt], axis=-1)
    s1 = jnp.concatenate([nh, nt], axis=-1)
    s2 = jnp.concatenate([pr, nr], axis=-1)
    idx_all = jnp.stack([s0, s1, s2], axis=2)
    return _transe_loss(entity_emb, relation_emb, idx_all)
